# baseline (speedup 1.0000x reference)
import functools

import jax
import jax.numpy as jnp
from jax import lax
from jax.experimental import pallas as pl
from jax.experimental.pallas import tpu as pltpu
from jax.experimental.pallas import tpu_sc as plsc

D = 768
P = 8
E = 4096

NC = 2
NS = 16
NW = NC * NS
L = 16
CH = E // NW
NV = CH // L
SUB = 32
NSUB = CH // SUB

BR = 128
E_PAD = E + P * BR
NB = E_PAD // BR
NBPAD = 64

_i32 = jnp.int32
_f32 = jnp.float32


def _dispatch_body(idx_hbm, zs_hbm, zd_hbm,
                   zs_out, zd_out, dpos_out, blk_out,
                   idx_v, myidx_v, dpos2d_v, dposf_v, blk_v, rows_v,
                   sem_in, sem_out):
    wid = lax.axis_index("s") * NC + lax.axis_index("c")
    base = wid * CH

    pltpu.sync_copy(idx_hbm, idx_v)
    pltpu.sync_copy(idx_hbm.at[pl.ds(base, CH)], myidx_v)

    my_first_vreg = wid * NV

    def hist_step(i, carry):
        tot, bef = carry
        v = idx_v[pl.ds(i * L, L)]
        flagv = jnp.where(
            jnp.full((L,), 0, _i32) + i < my_first_vreg, 1, 0)
        tot2, bef2 = [], []
        for p in range(P):
            o = jnp.where(v == p, 1, 0)
            tot2.append(tot[p] + o)
            bef2.append(bef[p] + o * flagv)
        return tuple(tot2), tuple(bef2)

    zeros = tuple(jnp.zeros((L,), _i32) for _ in range(P))
    tot, bef = lax.fori_loop(0, E // L, hist_step, (zeros, zeros))

    tot_s = [jnp.sum(tot[p]) for p in range(P)]
    bef_s = [jnp.sum(bef[p]) for p in range(P)]
    pad_s = [((t + (BR - 1)) // BR) * BR for t in tot_s]
    seg = []
    run = jnp.int32(0)
    for p in range(P):
        seg.append(run)
        run = run + pad_s[p]

    base_run = [seg[p] + bef_s[p] for p in range(P)]
    for j in range(NV):
        v = myidx_v[pl.ds(j * L, L)]
        dst = jnp.zeros((L,), _i32)
        for p in range(P):
            m = v == p
            ones = jnp.where(m, 1, 0)
            incl = plsc.cumsum(ones)
            pos = base_run[p] + incl - 1
            dst = jnp.where(m, pos, dst)
            base_run[p] = base_run[p] + jnp.sum(ones)
        dposf_v[pl.ds(j * L, L)] = dst
        dpos2d_v[j // 2, pl.ds((j % 2) * L, L)] = dst

    pltpu.sync_copy(dposf_v, dpos_out.at[pl.ds(base, CH)])

    @pl.when(wid == 0)
    def _():
        for t in range(NBPAD // L):
            b_ids = lax.iota(_i32, L) + t * L
            cnt = jnp.zeros((L,), _i32)
            for p in range(P):
                cnt = cnt + jnp.where(b_ids >= seg[p] // BR, 1, 0)
            blk_v[pl.ds(t * L, L)] = jnp.minimum(cnt - 1, P - 1)
        pltpu.sync_copy(blk_v, blk_out)

    for src_hbm, dst_hbm in ((zs_hbm, zs_out), (zd_hbm, zd_out)):
        cps_in = []
        cps_out = []
        for j in range(NSUB):
            cin = pltpu.make_async_copy(
                src_hbm.at[pl.ds(base + j * SUB, SUB)], rows_v.at[j % 2],
                sem_in)
            cout = pltpu.make_async_copy(
                rows_v.at[j % 2], dst_hbm.at[dpos2d_v.at[j]], sem_out)
            cps_in.append(cin)
            cps_out.append(cout)
        cps_in[0].start()
        for j in range(NSUB):
            cps_in[j].wait()
            cps_out[j].start()
            if j + 1 < NSUB:
                if j >= 1:
                    cps_out[j - 1].wait()
                cps_in[j + 1].start()
        cps_out[NSUB - 2].wait()
        cps_out[NSUB - 1].wait()


def _gather_body(scores_hbm, dpos_hbm, out_hbm, sc_v, mypos_v, outb_v, junk):
    wid = lax.axis_index("s") * NC + lax.axis_index("c")
    base = wid * CH
    pltpu.sync_copy(scores_hbm, sc_v)
    pltpu.sync_copy(dpos_hbm.at[pl.ds(base, CH)], mypos_v)
    for j in range(NV):
        iv = mypos_v[pl.ds(j * L, L)]
        outb_v[pl.ds(j * L, L)] = plsc.load_gather(sc_v, [iv])
    pltpu.sync_copy(outb_v, out_hbm.at[pl.ds(base, CH)])


def _grouped_body(s_ref, zs_ref, zd_ref, wp_ref, bp_ref, wb_ref, bb_ref,
                  out_ref):
    zs = zs_ref[...]
    zd = zd_ref[...]
    u = lax.dot_general(zd, wb_ref[0], (((1,), (1,)), ((), ())),
                        preferred_element_type=_f32)
    prj = lax.dot_general(zs, wp_ref[0], (((1,), (1,)), ((), ())),
                          preferred_element_type=_f32)
    prj = prj + bp_ref[0, 0][None, :]
    s = jnp.sum(prj * u, axis=1, keepdims=True)
    out_ref[...] = s + bb_ref[0, 0]


_sc_mesh = plsc.VectorSubcoreMesh(core_axis_name="c", subcore_axis_name="s")
_sc_params = pltpu.CompilerParams(needs_layout_passes=False)

_dispatch = functools.partial(
    pl.kernel,
    out_type=[
        jax.ShapeDtypeStruct((E_PAD, D), _f32),
        jax.ShapeDtypeStruct((E_PAD, D), _f32),
        jax.ShapeDtypeStruct((E,), _i32),
        jax.ShapeDtypeStruct((NBPAD,), _i32),
    ],
    mesh=_sc_mesh,
    scratch_types=[
        pltpu.VMEM((E,), _i32),
        pltpu.VMEM((CH,), _i32),
        pltpu.VMEM((NSUB, SUB), _i32),
        pltpu.VMEM((CH,), _i32),
        pltpu.VMEM((NBPAD,), _i32),
        pltpu.VMEM((2, SUB, D), _f32),
        pltpu.SemaphoreType.DMA,
        pltpu.SemaphoreType.DMA,
    ],
    compiler_params=_sc_params,
)(_dispatch_body)

_gather = functools.partial(
    pl.kernel,
    out_type=jax.ShapeDtypeStruct((E,), _f32),
    mesh=_sc_mesh,
    scratch_types=[
        pltpu.VMEM((E_PAD,), _f32),
        pltpu.VMEM((CH,), _i32),
        pltpu.VMEM((CH,), _f32),
        pltpu.SemaphoreType.DMA,
    ],
    compiler_params=_sc_params,
)(_gather_body)


def kernel(z_src, z_dst, lr_pair_idx, W_proj, b_proj, W_bil, b_bil):
    idx = lr_pair_idx.astype(_i32)
    bb = b_bil.astype(_f32).reshape(1, 1)

    zs_sorted, zd_sorted, dst_pos, blk_expert = _dispatch(idx, z_src, z_dst)

    grid_spec = pltpu.PrefetchScalarGridSpec(
        num_scalar_prefetch=1,
        grid=(NB,),
        in_specs=[
            pl.BlockSpec((BR, D), lambda b, s: (b, 0)),
            pl.BlockSpec((BR, D), lambda b, s: (b, 0)),
            pl.BlockSpec((1, D, D), lambda b, s: (s[b], 0, 0)),
            pl.BlockSpec((1, 1, D), lambda b, s: (s[b], 0, 0)),
            pl.BlockSpec((1, D, D), lambda b, s: (0, 0, 0)),
            pl.BlockSpec(memory_space=pltpu.SMEM),
        ],
        out_specs=pl.BlockSpec((BR, 1), lambda b, s: (b, 0)),
    )
    scores_sorted = pl.pallas_call(
        _grouped_body,
        grid_spec=grid_spec,
        out_shape=jax.ShapeDtypeStruct((E_PAD, 1), _f32),
    )(blk_expert, zs_sorted, zd_sorted, W_proj, b_proj.reshape(P, 1, D),
      W_bil, bb)

    scores = _gather(scores_sorted.reshape(E_PAD), dst_pos)
    return scores.reshape(E, 1)


# speedup vs baseline: 1.3255x; 1.3255x over previous
import functools

import jax
import jax.numpy as jnp
from jax import lax
from jax.experimental import pallas as pl
from jax.experimental.pallas import tpu as pltpu
from jax.experimental.pallas import tpu_sc as plsc

D = 768
P = 8
E = 4096

NC = 2
NS = 16
NW = NC * NS
L = 16
CH = E // NW
NV = CH // L
SUB = 32
NSUB = CH // SUB
NSTEP = 2 * NSUB
NBUF = 4

BR = 256
E_PAD = E + P * BR
NB = E_PAD // BR
NBPAD = 32

_i32 = jnp.int32
_f32 = jnp.float32


def _dispatch_body(idx_hbm, zs_hbm, zd_hbm,
                   zs_out, zd_out, dpos_out, blk_out,
                   idx_v, myidx_v, dpos2d_v, dposf_v, blk_v, rows_v,
                   sin0, sin1, sin2, sin3, sout0, sout1, sout2, sout3):
    wid = lax.axis_index("s") * NC + lax.axis_index("c")
    base = wid * CH
    sins = [sin0, sin1, sin2, sin3]
    souts = [sout0, sout1, sout2, sout3]

    def src_of(k):
        return zs_hbm if k < NSUB else zd_hbm

    def dst_of(k):
        return zs_out if k < NSUB else zd_out

    def sub_of(k):
        return k % NSUB

    cps_in = []
    cps_out = []
    for k in range(NSTEP):
        j = sub_of(k)
        b = k % NBUF
        cps_in.append(pltpu.make_async_copy(
            src_of(k).at[pl.ds(base + j * SUB, SUB)], rows_v.at[b], sins[b]))
        cps_out.append(pltpu.make_async_copy(
            rows_v.at[b], dst_of(k).at[dpos2d_v.at[j]], souts[b]))

    for k in range(NBUF):
        cps_in[k].start()

    pltpu.sync_copy(idx_hbm, idx_v)
    pltpu.sync_copy(idx_hbm.at[pl.ds(base, CH)], myidx_v)

    my_first_vreg = wid * NV

    def hist_step(i, carry):
        tot, bef = carry
        v = idx_v[pl.ds(i * L, L)]
        flagv = jnp.where(jnp.full((L,), 0, _i32) + i < my_first_vreg, 1, 0)
        tot2, bef2 = [], []
        for p in range(P):
            o = jnp.where(v == p, 1, 0)
            tot2.append(tot[p] + o)
            bef2.append(bef[p] + o * flagv)
        return tuple(tot2), tuple(bef2)

    zeros = tuple(jnp.zeros((L,), _i32) for _ in range(P))
    tot, bef = lax.fori_loop(0, E // L, hist_step, (zeros, zeros))

    tot_s = [jnp.sum(tot[p]) for p in range(P)]
    bef_s = [jnp.sum(bef[p]) for p in range(P)]
    pad_s = [((t + (BR - 1)) // BR) * BR for t in tot_s]
    seg = []
    run = jnp.int32(0)
    for p in range(P):
        seg.append(run)
        run = run + pad_s[p]

    base_run = [seg[p] + bef_s[p] for p in range(P)]
    for j in range(NV):
        v = myidx_v[pl.ds(j * L, L)]
        dst = jnp.zeros((L,), _i32)
        for p in range(P):
            m = v == p
            ones = jnp.where(m, 1, 0)
            incl = plsc.cumsum(ones)
            pos = base_run[p] + incl - 1
            dst = jnp.where(m, pos, dst)
            base_run[p] = base_run[p] + jnp.sum(ones)
        dposf_v[pl.ds(j * L, L)] = dst
        dpos2d_v[j // 2, pl.ds((j % 2) * L, L)] = dst

    for k in range(NSTEP):
        cps_in[k].wait()
        cps_out[k].start()
        if k >= 1 and k + NBUF - 1 < NSTEP:
            cps_out[k - 1].wait()
            cps_in[k + NBUF - 1].start()
    for k in range(NSTEP - NBUF, NSTEP):
        cps_out[k].wait()

    pltpu.sync_copy(dposf_v, dpos_out.at[pl.ds(base, CH)])

    @pl.when(wid == 0)
    def _():
        for t in range(NBPAD // L):
            b_ids = lax.iota(_i32, L) + t * L
            cnt = jnp.zeros((L,), _i32)
            for p in range(P):
                cnt = cnt + jnp.where(b_ids >= seg[p] // BR, 1, 0)
            blk_v[0, pl.ds(t * L, L)] = jnp.minimum(cnt - 1, P - 1)
            blk_v[1, pl.ds(t * L, L)] = jnp.where(b_ids * BR < run, 1, 0)
        pltpu.sync_copy(blk_v, blk_out)


def _gather_body(scores_hbm, dpos_hbm, out_hbm, sc_v, mypos_v, outb_v, junk):
    wid = lax.axis_index("s") * NC + lax.axis_index("c")
    base = wid * CH
    pltpu.sync_copy(scores_hbm, sc_v)
    pltpu.sync_copy(dpos_hbm.at[pl.ds(base, CH)], mypos_v)
    zl = jnp.zeros((L,), _i32)
    for j in range(NV):
        iv = mypos_v[pl.ds(j * L, L)]
        outb_v[pl.ds(j * L, L)] = plsc.load_gather(sc_v, [zl, iv])
    pltpu.sync_copy(outb_v, out_hbm.at[pl.ds(base, CH)])


def _grouped_body(s_ref, zs_ref, zd_ref, wp_ref, bp_ref, wb_ref, bb_ref,
                  out_ref):
    @pl.when(s_ref[1, pl.program_id(0)] == 1)
    def _():
        zs = zs_ref[...]
        zd = zd_ref[...]
        u = lax.dot_general(zd, wb_ref[0], (((1,), (1,)), ((), ())),
                            preferred_element_type=_f32)
        prj = lax.dot_general(zs, wp_ref[0], (((1,), (1,)), ((), ())),
                              preferred_element_type=_f32)
        prj = prj + bp_ref[0, 0][None, :]
        s = jnp.sum(prj * u, axis=1, keepdims=True)
        out_ref[...] = (s + bb_ref[0, 0]).reshape(1, BR)


_sc_mesh = plsc.VectorSubcoreMesh(core_axis_name="c", subcore_axis_name="s")
_sc_params = pltpu.CompilerParams(needs_layout_passes=False)

_dispatch = functools.partial(
    pl.kernel,
    out_type=[
        jax.ShapeDtypeStruct((E_PAD, D), _f32),
        jax.ShapeDtypeStruct((E_PAD, D), _f32),
        jax.ShapeDtypeStruct((E,), _i32),
        jax.ShapeDtypeStruct((2, NBPAD), _i32),
    ],
    mesh=_sc_mesh,
    scratch_types=[
        pltpu.VMEM((E,), _i32),
        pltpu.VMEM((CH,), _i32),
        pltpu.VMEM((NSUB, SUB), _i32),
        pltpu.VMEM((CH,), _i32),
        pltpu.VMEM((2, NBPAD), _i32),
        pltpu.VMEM((NBUF, SUB, D), _f32),
    ] + [pltpu.SemaphoreType.DMA] * (2 * NBUF),
    compiler_params=_sc_params,
)(_dispatch_body)

_gather = functools.partial(
    pl.kernel,
    out_type=jax.ShapeDtypeStruct((E,), _f32),
    mesh=_sc_mesh,
    scratch_types=[
        pltpu.VMEM((1, E_PAD), _f32),
        pltpu.VMEM((CH,), _i32),
        pltpu.VMEM((CH,), _f32),
        pltpu.SemaphoreType.DMA,
    ],
    compiler_params=_sc_params,
)(_gather_body)


def kernel(z_src, z_dst, lr_pair_idx, W_proj, b_proj, W_bil, b_bil):
    idx = lr_pair_idx.astype(_i32)
    bb = b_bil.astype(_f32).reshape(1, 1)

    zs_sorted, zd_sorted, dst_pos, blk_tab = _dispatch(idx, z_src, z_dst)

    grid_spec = pltpu.PrefetchScalarGridSpec(
        num_scalar_prefetch=1,
        grid=(NB,),
        in_specs=[
            pl.BlockSpec((BR, D), lambda b, s: (jnp.where(s[1, b] == 1, b, 0), 0)),
            pl.BlockSpec((BR, D), lambda b, s: (jnp.where(s[1, b] == 1, b, 0), 0)),
            pl.BlockSpec((1, D, D), lambda b, s: (s[0, b], 0, 0)),
            pl.BlockSpec((1, 1, D), lambda b, s: (s[0, b], 0, 0)),
            pl.BlockSpec((1, D, D), lambda b, s: (0, 0, 0)),
            pl.BlockSpec(memory_space=pltpu.SMEM),
        ],
        out_specs=pl.BlockSpec((1, BR), lambda b, s: (0, b)),
    )
    scores_sorted = pl.pallas_call(
        _grouped_body,
        grid_spec=grid_spec,
        out_shape=jax.ShapeDtypeStruct((1, E_PAD), _f32),
    )(blk_tab, zs_sorted, zd_sorted, W_proj, b_proj.reshape(P, 1, D),
      W_bil, bb)

    scores = _gather(scores_sorted, dst_pos)
    return scores.reshape(E, 1)


# speedup vs baseline: 1.3394x; 1.0104x over previous
import functools

import jax
import jax.numpy as jnp
from jax import lax
from jax.experimental import pallas as pl
from jax.experimental.pallas import tpu as pltpu
from jax.experimental.pallas import tpu_sc as plsc

D = 768
P = 8
E = 4096

NC = 2
NS = 16
NW = NC * NS
L = 16
CH = E // NW
NV = CH // L
SUB = 32
NSUB = CH // SUB
NSTEP = 2 * NSUB
NBUF = 4

BR = 256
E_PAD = E + P * BR
NB = E_PAD // BR
NBPAD = 32

_i32 = jnp.int32
_f32 = jnp.float32


def _dispatch_body(idx_hbm, zs_hbm, zd_hbm,
                   zs_out, zd_out, dpos_out, blk_out,
                   idx_v, myidx_v, dpos2d_v, dposf_v, blk_v, rows_v,
                   sin0, sin1, sin2, sin3, sout0, sout1, sout2, sout3):
    wid = lax.axis_index("s") * NC + lax.axis_index("c")
    base = wid * CH
    sins = [sin0, sin1, sin2, sin3]
    souts = [sout0, sout1, sout2, sout3]

    def src_of(k):
        return zs_hbm if k < NSUB else zd_hbm

    def dst_of(k):
        return zs_out if k < NSUB else zd_out

    def sub_of(k):
        return k % NSUB

    cps_in = []
    cps_out = []
    for k in range(NSTEP):
        j = sub_of(k)
        b = k % NBUF
        cps_in.append(pltpu.make_async_copy(
            src_of(k).at[pl.ds(base + j * SUB, SUB)], rows_v.at[b], sins[b]))
        cps_out.append(pltpu.make_async_copy(
            rows_v.at[b], dst_of(k).at[dpos2d_v.at[j]], souts[b]))

    for k in range(NBUF):
        cps_in[k].start()

    pltpu.sync_copy(idx_hbm, idx_v)
    pltpu.sync_copy(idx_hbm.at[pl.ds(base, CH)], myidx_v)

    my_first_vreg = wid * NV

    def count_step(i, carry):
        v = idx_v[pl.ds(i * L, L)]
        return tuple(carry[p] + jnp.where(v == p, 1, 0) for p in range(P))

    zeros = tuple(jnp.zeros((L,), _i32) for _ in range(P))
    bef = lax.fori_loop(0, my_first_vreg, count_step, zeros)
    aft = lax.fori_loop(my_first_vreg + NV, E // L, count_step, zeros)

    my = zeros
    for j in range(NV):
        v = myidx_v[pl.ds(j * L, L)]
        my = tuple(my[p] + jnp.where(v == p, 1, 0) for p in range(P))

    bef_s = [jnp.sum(bef[p]) for p in range(P)]
    tot_s = [bef_s[p] + jnp.sum(my[p]) + jnp.sum(aft[p]) for p in range(P)]
    pad_s = [((t + (BR - 1)) // BR) * BR for t in tot_s]
    seg = []
    run = jnp.int32(0)
    for p in range(P):
        seg.append(run)
        run = run + pad_s[p]

    base_run = [seg[p] + bef_s[p] for p in range(P)]
    for j in range(NV):
        v = myidx_v[pl.ds(j * L, L)]
        dst = jnp.zeros((L,), _i32)
        for p in range(P):
            m = v == p
            ones = jnp.where(m, 1, 0)
            incl = plsc.cumsum(ones)
            pos = base_run[p] + incl - 1
            dst = jnp.where(m, pos, dst)
            base_run[p] = base_run[p] + jnp.sum(ones)
        dposf_v[pl.ds(j * L, L)] = dst
        dpos2d_v[j // 2, pl.ds((j % 2) * L, L)] = dst

    for k in range(NBUF):
        cps_in[k].wait()
        cps_out[k].start()
    for k in range(NBUF):
        cps_out[k].wait()
        cps_in[k + NBUF].start()
    for k in range(NBUF, NSTEP):
        cps_in[k].wait()
        cps_out[k].start()
    for k in range(NBUF, NSTEP):
        cps_out[k].wait()

    pltpu.sync_copy(dposf_v, dpos_out.at[pl.ds(base, CH)])

    @pl.when(wid == 0)
    def _():
        for t in range(NBPAD // L):
            b_ids = lax.iota(_i32, L) + t * L
            cnt = jnp.zeros((L,), _i32)
            for p in range(P):
                cnt = cnt + jnp.where(b_ids >= seg[p] // BR, 1, 0)
            blk_v[0, pl.ds(t * L, L)] = jnp.minimum(cnt - 1, P - 1)
            blk_v[1, pl.ds(t * L, L)] = jnp.where(b_ids * BR < run, 1, 0)
        pltpu.sync_copy(blk_v, blk_out)


def _gather_body(scores_hbm, dpos_hbm, out_hbm, sc_v, mypos_v, outb_v, junk):
    wid = lax.axis_index("s") * NC + lax.axis_index("c")
    base = wid * CH
    pltpu.sync_copy(scores_hbm, sc_v)
    pltpu.sync_copy(dpos_hbm.at[pl.ds(base, CH)], mypos_v)
    zl = jnp.zeros((L,), _i32)
    for j in range(NV):
        iv = mypos_v[pl.ds(j * L, L)]
        outb_v[pl.ds(j * L, L)] = plsc.load_gather(sc_v, [zl, iv])
    pltpu.sync_copy(outb_v, out_hbm.at[pl.ds(base, CH)])


def _grouped_body(s_ref, zs_ref, zd_ref, wp_ref, bp_ref, wb_ref, bb_ref,
                  out_ref):
    b = pl.program_id(0)

    @pl.when(s_ref[1, b] == 1)
    def _():
        g = s_ref[0, b]
        zs = zs_ref[...]
        zd = zd_ref[...]
        wg = wp_ref[pl.ds(g, 1)][0]
        u = lax.dot_general(zd, wb_ref[0], (((1,), (1,)), ((), ())),
                            preferred_element_type=_f32)
        prj = lax.dot_general(zs, wg, (((1,), (1,)), ((), ())),
                              preferred_element_type=_f32)
        prj = prj + bp_ref[pl.ds(g, 1), 0][0][None, :]
        s = jnp.sum(prj * u, axis=1, keepdims=True)
        out_ref[...] = (s + bb_ref[0, 0]).reshape(1, BR)


_sc_mesh = plsc.VectorSubcoreMesh(core_axis_name="c", subcore_axis_name="s")
_sc_params = pltpu.CompilerParams(needs_layout_passes=False)

_dispatch = functools.partial(
    pl.kernel,
    out_type=[
        jax.ShapeDtypeStruct((E_PAD, D), _f32),
        jax.ShapeDtypeStruct((E_PAD, D), _f32),
        jax.ShapeDtypeStruct((E,), _i32),
        jax.ShapeDtypeStruct((2, NBPAD), _i32),
    ],
    mesh=_sc_mesh,
    scratch_types=[
        pltpu.VMEM((E,), _i32),
        pltpu.VMEM((CH,), _i32),
        pltpu.VMEM((NSUB, SUB), _i32),
        pltpu.VMEM((CH,), _i32),
        pltpu.VMEM((2, NBPAD), _i32),
        pltpu.VMEM((NBUF, SUB, D), _f32),
    ] + [pltpu.SemaphoreType.DMA] * (2 * NBUF),
    compiler_params=_sc_params,
)(_dispatch_body)

_gather = functools.partial(
    pl.kernel,
    out_type=jax.ShapeDtypeStruct((E,), _f32),
    mesh=_sc_mesh,
    scratch_types=[
        pltpu.VMEM((1, E_PAD), _f32),
        pltpu.VMEM((CH,), _i32),
        pltpu.VMEM((CH,), _f32),
        pltpu.SemaphoreType.DMA,
    ],
    compiler_params=_sc_params,
)(_gather_body)


def kernel(z_src, z_dst, lr_pair_idx, W_proj, b_proj, W_bil, b_bil):
    idx = lr_pair_idx.astype(_i32)
    bb = b_bil.astype(_f32).reshape(1, 1)

    zs_sorted, zd_sorted, dst_pos, blk_tab = _dispatch(idx, z_src, z_dst)

    grid_spec = pltpu.PrefetchScalarGridSpec(
        num_scalar_prefetch=1,
        grid=(NB,),
        in_specs=[
            pl.BlockSpec((BR, D), lambda b, s: (jnp.where(s[1, b] == 1, b, 0), 0)),
            pl.BlockSpec((BR, D), lambda b, s: (jnp.where(s[1, b] == 1, b, 0), 0)),
            pl.BlockSpec((P, D, D), lambda b, s: (0, 0, 0)),
            pl.BlockSpec((P, 1, D), lambda b, s: (0, 0, 0)),
            pl.BlockSpec((1, D, D), lambda b, s: (0, 0, 0)),
            pl.BlockSpec(memory_space=pltpu.SMEM),
        ],
        out_specs=pl.BlockSpec((1, BR), lambda b, s: (0, b)),
    )
    scores_sorted = pl.pallas_call(
        _grouped_body,
        grid_spec=grid_spec,
        out_shape=jax.ShapeDtypeStruct((1, E_PAD), _f32),
    )(blk_tab, zs_sorted, zd_sorted, W_proj, b_proj.reshape(P, 1, D),
      W_bil, bb)

    scores = _gather(scores_sorted, dst_pos)
    return scores.reshape(E, 1)
